# baseline (speedup 1.0000x reference)
import jax
import jax.numpy as jnp
from jax.experimental import pallas as pl
from jax.experimental.pallas import tpu as pltpu

K = 32
CB = 128
SS = 8


def _sum_body(x_ref, out_ref):
    s = pl.program_id(1)
    part = jnp.sum(x_ref[...], axis=(0, 2))[None, :]

    @pl.when(s == 0)
    def _():
        out_ref[...] = part

    @pl.when(s != 0)
    def _():
        out_ref[...] += part


def _topk_body(sums_ref, idx_ref):
    vals = sums_ref[...]
    c = vals.shape[1]
    iota = jax.lax.broadcasted_iota(jnp.int32, vals.shape, 1)
    kiota = jax.lax.broadcasted_iota(jnp.int32, (1, K), 1)

    def body(j, carry):
        v, idxs = carry
        m = jnp.max(v)
        am = jnp.min(jnp.where(v == m, iota, c))
        idxs = jnp.where(kiota == j, am, idxs)
        v = jnp.where(iota == am, -jnp.inf, v)
        return v, idxs

    _, idxs = jax.lax.fori_loop(
        0, K, body, (vals, jnp.zeros((1, K), jnp.int32)))
    idx_ref[...] = idxs


def _gather_body(idx_ref, x_ref, out_ref):
    del idx_ref
    out_ref[...] = x_ref[...]


def kernel(x):
    b, c, h, w = x.shape
    s = h * w
    x3 = x.reshape(b, c, s)

    sums = pl.pallas_call(
        _sum_body,
        grid=(c // CB, SS),
        in_specs=[pl.BlockSpec(
            (b, CB, s // SS), lambda j, t: (0, j, t))],
        out_specs=pl.BlockSpec((1, CB), lambda j, t: (0, j)),
        out_shape=jax.ShapeDtypeStruct((1, c), jnp.float32),
    )(x3)

    idx = pl.pallas_call(
        _topk_body,
        out_shape=jax.ShapeDtypeStruct((1, K), jnp.int32),
    )(sums)[0]

    x4 = x.reshape(b, c, s // 128, 128)
    out = pl.pallas_call(
        _gather_body,
        grid_spec=pltpu.PrefetchScalarGridSpec(
            num_scalar_prefetch=1,
            grid=(K,),
            in_specs=[pl.BlockSpec(
                (b, 1, s // 128, 128),
                lambda j, idx_ref: (0, idx_ref[j], 0, 0))],
            out_specs=pl.BlockSpec(
                (b, 1, s // 128, 128), lambda j, idx_ref: (0, j, 0, 0)),
        ),
        out_shape=jax.ShapeDtypeStruct((b, K, s // 128, 128), jnp.float32),
    )(idx, x4)
    return out.reshape(b, K, h, w)
